# baseline (speedup 1.0000x reference)
import functools

import jax
import jax.numpy as jnp
from jax import lax
from jax.experimental import pallas as pl
from jax.experimental.pallas import tpu as pltpu
from jax.experimental.pallas import tpu_sc as plsc

BATCH = 64
NUM_NODES = 512
N_USER = 256
SEQ_LEN = 256
D_HIDDEN = 512
K = 16
N_ROWS = BATCH * N_USER


def _nt_dot(a, b):
    return lax.dot_general(a, b, (((1,), (1,)), ((), ())),
                           preferred_element_type=jnp.float32)


def _nn_dot(a, b):
    return lax.dot_general(a, b, (((1,), (0,)), ((), ())),
                           preferred_element_type=jnp.float32)


def _prep_body(emb_ref, eg_ref, eb_ref, vwu_ref, vbu_ref, vgu_ref, vbtu_ref,
               vwi_ref, vbi_ref, vgi_ref, vbti_ref,
               num_ref, xn_ref, hvu_ref, hvi_ref):
    e = emb_ref[...]
    mean = jnp.mean(e, axis=0, keepdims=True)
    var = jnp.mean((e - mean) ** 2, axis=0, keepdims=True)
    v = (e - mean) / jnp.sqrt(var + 1e-5) * eg_ref[...] + eb_ref[...]
    xn_ref[...] = jnp.sqrt(jnp.sum(v * v, axis=1, keepdims=True)) + 1e-8
    vb = v.astype(jnp.bfloat16)
    num_ref[...] = lax.dot_general(vb, vb, (((1,), (1,)), ((), ())),
                                   preferred_element_type=jnp.float32)

    def bn_proj(vpart, w_ref, b_ref, g_ref, bt_ref, out_ref):
        t = _nt_dot(vpart, w_ref[...]) + b_ref[...]
        m = jnp.mean(t, axis=0, keepdims=True)
        va = jnp.mean((t - m) ** 2, axis=0, keepdims=True)
        out_ref[...] = (t - m) / jnp.sqrt(va + 1e-5) * g_ref[...] + bt_ref[...]

    bn_proj(v[:N_USER], vwu_ref, vbu_ref, vgu_ref, vbtu_ref, hvu_ref)
    bn_proj(v[N_USER:], vwi_ref, vbi_ref, vgi_ref, vbti_ref, hvi_ref)


_L = 16
_NW = 32
_ROWS_PER_W = NUM_NODES // _NW
_NCHUNK = NUM_NODES // _L

_sc_mesh = plsc.VectorSubcoreMesh(core_axis_name="c", subcore_axis_name="s")

_GDN = lax.GatherDimensionNumbers(offset_dims=(), collapsed_slice_dims=(0,),
                                  start_index_map=(0,))


def _perm(x, idx):
    return lax.gather(x, idx.reshape(_L, 1), _GDN, (1,),
                      mode=lax.GatherScatterMode.PROMISE_IN_BOUNDS)


def _allreduce_argmax(val, idx, lane):
    for d in (1, 2, 4, 8):
        p = lane ^ d
        oval = _perm(val, p)
        oidx = _perm(idx, p)
        take = (oval > val) | ((oval == val) & (oidx < idx))
        val = jnp.where(take, oval, val)
        idx = jnp.where(take, oidx, idx)
    return val, idx


@functools.partial(
    pl.kernel, mesh=_sc_mesh,
    out_type=jax.ShapeDtypeStruct((2, NUM_NODES, K), jnp.int32),
    scratch_types=[
        pltpu.VMEM((_ROWS_PER_W, NUM_NODES), jnp.float32),
        pltpu.VMEM((NUM_NODES,), jnp.float32),
        pltpu.VMEM((_L,), jnp.float32),
        pltpu.VMEM((NUM_NODES,), jnp.float32),
        pltpu.VMEM((_ROWS_PER_W, K), jnp.int32),
        pltpu.VMEM((_ROWS_PER_W, K), jnp.int32),
    ],
)
def _topk_sc(num_hbm, xn_hbm, src_hbm, edges_hbm, rows_v, xnall_v, xnrow_v,
             simrow_v, nbr_v, src_v):
    wid = lax.axis_index("s") * 2 + lax.axis_index("c")
    base = wid * _ROWS_PER_W
    pltpu.sync_copy(num_hbm.at[pl.ds(base, _ROWS_PER_W)], rows_v)
    pltpu.sync_copy(xn_hbm, xnall_v)
    pltpu.sync_copy(xn_hbm.at[pl.ds(base, _ROWS_PER_W)], xnrow_v)
    pltpu.sync_copy(src_hbm.at[pl.ds(base, _ROWS_PER_W)], src_v)
    lane = lax.iota(jnp.int32, _L)
    neg = jnp.full((_L,), -jnp.inf, jnp.float32)

    for r in range(_ROWS_PER_W):
        r_idx = jnp.full((_L,), r, jnp.int32)
        xn_i = _perm(xnrow_v[...], r_idx)

        def chunk_div(j, _c, r=r, xn_i=xn_i):
            nc = rows_v[r, pl.ds(j * _L, _L)]
            xj = xnall_v[pl.ds(j * _L, _L)]
            simrow_v[pl.ds(j * _L, _L)] = nc / (xn_i * xj)
            return 0

        lax.fori_loop(0, _NCHUNK, chunk_div, 0)

        def k_body(k, carry, lane=lane, neg=neg):
            nbrvec, kvec, lastv, lasti = carry

            def scan_chunk(j, car, lastv=lastv, lasti=lasti):
                am, ai, jb = car
                cv = simrow_v[pl.ds(j * _L, _L)]
                elig = (cv < lastv) | ((cv == lastv) & (jb > lasti))
                upd = elig & (cv > am)
                am = jnp.where(upd, cv, am)
                ai = jnp.where(upd, jb, ai)
                return (am, ai, jb + _L)

            am, ai, _ = lax.fori_loop(
                0, _NCHUNK, scan_chunk,
                (neg, jnp.full((_L,), 1 << 30, jnp.int32), lane))
            gval, gidx = _allreduce_argmax(am, ai, lane)
            nbrvec = jnp.where(lane == kvec, gidx, nbrvec)
            return (nbrvec, kvec + 1, gval, gidx)

        nbrvec, _, _, _ = lax.fori_loop(
            0, K, k_body,
            (jnp.zeros((_L,), jnp.int32), jnp.zeros((_L,), jnp.int32),
             jnp.full((_L,), jnp.inf, jnp.float32),
             jnp.full((_L,), -1, jnp.int32)))
        nbr_v[r, :] = nbrvec

    pltpu.sync_copy(src_v, edges_hbm.at[0, pl.ds(base, _ROWS_PER_W)])
    pltpu.sync_copy(nbr_v, edges_hbm.at[1, pl.ds(base, _ROWS_PER_W)])


_BB = 16
_NSTEP = BATCH // _BB


def _fused_body(x_ref, wtu_ref, bu_ref, gu_ref, btu_ref, wti_ref, bi_ref,
                gi_ref, bti_ref, hvu_ref, hvi_ref,
                out_ref,
                mu_ref, su_ref, mi_ref, si_ref,
                wsu_ref, biasu_ref, wsi_ref, biasi_ref, xbf_ref):
    t = pl.program_id(0)
    rows = _BB * N_USER

    @pl.when(t < _NSTEP)
    def _stats():
        xu = x_ref[:, 0].reshape(rows, SEQ_LEN)
        xi = x_ref[:, 1].reshape(rows, SEQ_LEN)
        xub = xu.astype(jnp.bfloat16)
        xib = xi.astype(jnp.bfloat16)
        xbf_ref[0, pl.ds(t * rows, rows), :] = xub
        xbf_ref[1, pl.ds(t * rows, rows), :] = xib
        mu = lax.dot_general(xub, xub, (((0,), (0,)), ((), ())),
                             preferred_element_type=jnp.float32)
        mi = lax.dot_general(xib, xib, (((0,), (0,)), ((), ())),
                             preferred_element_type=jnp.float32)
        su = jnp.sum(xu, axis=0, keepdims=True)
        si = jnp.sum(xi, axis=0, keepdims=True)

        @pl.when(t == 0)
        def _():
            mu_ref[...] = mu
            su_ref[...] = su
            mi_ref[...] = mi
            si_ref[...] = si

        @pl.when(t != 0)
        def _():
            mu_ref[...] += mu
            su_ref[...] += su
            mi_ref[...] += mi
            si_ref[...] += si

        @pl.when(t == _NSTEP - 1)
        def _fold():
            eye = (lax.broadcasted_iota(jnp.int32, (D_HIDDEN, D_HIDDEN), 0)
                   == lax.broadcasted_iota(jnp.int32, (D_HIDDEN, D_HIDDEN),
                                           1)).astype(jnp.float32)

            def fold(w_ref, b_ref, g_ref, bt_ref, m_ref, s_ref,
                     ws_ref, bias_ref):
                w = w_ref[...]
                bb = b_ref[...]
                g = g_ref[...]
                bt = bt_ref[...]
                ssum = s_ref[...]
                sh = _nt_dot(w, ssum)
                meanh = sh / N_ROWS + bb
                t2 = _nn_dot(w, m_ref[...])
                q = jnp.sum(t2 * w, axis=1, keepdims=True)
                varh = (q + 2.0 * bb * sh) / N_ROWS + bb * bb - meanh * meanh
                sc = g / jnp.sqrt(varh + 1e-5)
                ws_ref[...] = (w * sc).astype(jnp.bfloat16)
                bias_col = (bb - meanh) * sc + bt
                bias_ref[...] = lax.dot_general(
                    bias_col, eye, (((0,), (0,)), ((), ())),
                    preferred_element_type=jnp.float32)

            fold(wtu_ref, bu_ref, gu_ref, btu_ref, mu_ref, su_ref,
                 wsu_ref, biasu_ref)
            fold(wti_ref, bi_ref, gi_ref, bti_ref, mi_ref, si_ref,
                 wsi_ref, biasi_ref)

    @pl.when(t >= _NSTEP)
    def _main():
        s = (t - _NSTEP) % 4
        b = (t - _NSTEP) // 4

        @pl.when(s == 0)
        def _():
            xm = xbf_ref[0, pl.ds(b * rows, rows), :]
            out_ref[...] = _nt_dot(xm, wsu_ref[...]) + biasu_ref[...]

        @pl.when(s == 1)
        def _():
            hv = hvu_ref[...]
            for i in range(_BB):
                out_ref[pl.ds(i * N_USER, N_USER), :] = hv

        @pl.when(s == 2)
        def _():
            xm = xbf_ref[1, pl.ds(b * rows, rows), :]
            out_ref[...] = _nt_dot(xm, wsi_ref[...]) + biasi_ref[...]

        @pl.when(s == 3)
        def _():
            hv = hvi_ref[...]
            for i in range(_BB):
                out_ref[pl.ds(i * N_USER, N_USER), :] = hv


def kernel(x, emb_table, emb_gamma, emb_beta, xw_u, xb_u, xg_u, xbt_u, vw_u,
           vb_u, vg_u, vbt_u, xw_i, xb_i, xg_i, xbt_i, vw_i, vb_i, vg_i,
           vbt_i):
    f32 = jnp.float32
    row = lambda a: a.reshape(1, D_HIDDEN)
    colv = lambda a: a.reshape(D_HIDDEN, 1)

    num, xn, hv_u, hv_i = pl.pallas_call(
        _prep_body,
        out_shape=[
            jax.ShapeDtypeStruct((NUM_NODES, NUM_NODES), f32),
            jax.ShapeDtypeStruct((NUM_NODES, 1), f32),
            jax.ShapeDtypeStruct((N_USER, D_HIDDEN), f32),
            jax.ShapeDtypeStruct((N_USER, D_HIDDEN), f32),
        ],
    )(emb_table, row(emb_gamma), row(emb_beta),
      vw_u, row(vb_u), row(vg_u), row(vbt_u),
      vw_i, row(vb_i), row(vg_i), row(vbt_i))

    src_const = jnp.broadcast_to(
        jnp.arange(NUM_NODES, dtype=jnp.int32)[:, None], (NUM_NODES, K))
    edges3 = _topk_sc(num, xn.reshape(NUM_NODES), src_const)
    edges = edges3.reshape(2, NUM_NODES * K)

    x4 = x.reshape(BATCH, 2, N_USER, SEQ_LEN)
    n4 = 4 * _NSTEP
    cmap = lambda t: (0, 0)

    def x_map(t):
        return (jnp.minimum(t, _NSTEP - 1), 0, 0, 0)

    def out_map(t):
        tm = jnp.maximum(t - _NSTEP, 0)
        return (jnp.where(t < _NSTEP, 0, (tm % 4) * _NSTEP + tm // 4), 0)

    out = pl.pallas_call(
        _fused_body,
        grid=(_NSTEP + n4,),
        in_specs=[
            pl.BlockSpec((_BB, 2, N_USER, SEQ_LEN), x_map),
            pl.BlockSpec((D_HIDDEN, SEQ_LEN), cmap),
            pl.BlockSpec((D_HIDDEN, 1), cmap),
            pl.BlockSpec((D_HIDDEN, 1), cmap),
            pl.BlockSpec((D_HIDDEN, 1), cmap),
            pl.BlockSpec((D_HIDDEN, SEQ_LEN), cmap),
            pl.BlockSpec((D_HIDDEN, 1), cmap),
            pl.BlockSpec((D_HIDDEN, 1), cmap),
            pl.BlockSpec((D_HIDDEN, 1), cmap),
            pl.BlockSpec((N_USER, D_HIDDEN), cmap),
            pl.BlockSpec((N_USER, D_HIDDEN), cmap),
        ],
        out_specs=pl.BlockSpec((_BB * N_USER, D_HIDDEN), out_map),
        out_shape=jax.ShapeDtypeStruct((4 * N_ROWS, D_HIDDEN), f32),
        scratch_shapes=[
            pltpu.VMEM((SEQ_LEN, SEQ_LEN), f32),
            pltpu.VMEM((1, SEQ_LEN), f32),
            pltpu.VMEM((SEQ_LEN, SEQ_LEN), f32),
            pltpu.VMEM((1, SEQ_LEN), f32),
            pltpu.VMEM((D_HIDDEN, SEQ_LEN), jnp.bfloat16),
            pltpu.VMEM((1, D_HIDDEN), f32),
            pltpu.VMEM((D_HIDDEN, SEQ_LEN), jnp.bfloat16),
            pltpu.VMEM((1, D_HIDDEN), f32),
            pltpu.VMEM((2, N_ROWS, SEQ_LEN), jnp.bfloat16),
        ],
    )(x4, xw_u, colv(xb_u), colv(xg_u), colv(xbt_u),
      xw_i, colv(xb_i), colv(xg_i), colv(xbt_i),
      hv_u, hv_i)

    return (out, edges)


# speedup vs baseline: 1.0621x; 1.0621x over previous
import functools

import jax
import jax.numpy as jnp
from jax import lax
from jax.experimental import pallas as pl
from jax.experimental.pallas import tpu as pltpu
from jax.experimental.pallas import tpu_sc as plsc

BATCH = 64
NUM_NODES = 512
N_USER = 256
SEQ_LEN = 256
D_HIDDEN = 512
K = 16
N_ROWS = BATCH * N_USER


def _nt_dot(a, b):
    return lax.dot_general(a, b, (((1,), (1,)), ((), ())),
                           preferred_element_type=jnp.float32)


def _nn_dot(a, b):
    return lax.dot_general(a, b, (((1,), (0,)), ((), ())),
                           preferred_element_type=jnp.float32)


def _prep_body(emb_ref, eg_ref, eb_ref, vwu_ref, vbu_ref, vgu_ref, vbtu_ref,
               vwi_ref, vbi_ref, vgi_ref, vbti_ref,
               num_ref, xn_ref, hvu_ref, hvi_ref):
    e = emb_ref[...]
    mean = jnp.mean(e, axis=0, keepdims=True)
    var = jnp.mean((e - mean) ** 2, axis=0, keepdims=True)
    v = (e - mean) / jnp.sqrt(var + 1e-5) * eg_ref[...] + eb_ref[...]
    xn_ref[...] = jnp.sqrt(jnp.sum(v * v, axis=1, keepdims=True)) + 1e-8
    vb = v.astype(jnp.bfloat16)
    num_ref[...] = lax.dot_general(vb, vb, (((1,), (1,)), ((), ())),
                                   preferred_element_type=jnp.float32)

    def bn_proj(vpart, w_ref, b_ref, g_ref, bt_ref, out_ref):
        t = _nt_dot(vpart, w_ref[...]) + b_ref[...]
        m = jnp.mean(t, axis=0, keepdims=True)
        va = jnp.mean((t - m) ** 2, axis=0, keepdims=True)
        out_ref[...] = (t - m) / jnp.sqrt(va + 1e-5) * g_ref[...] + bt_ref[...]

    bn_proj(v[:N_USER], vwu_ref, vbu_ref, vgu_ref, vbtu_ref, hvu_ref)
    bn_proj(v[N_USER:], vwi_ref, vbi_ref, vgi_ref, vbti_ref, hvi_ref)


_L = 16
_NW = 32
_ROWS_PER_W = NUM_NODES // _NW
_NCHUNK = NUM_NODES // _L

_sc_mesh = plsc.VectorSubcoreMesh(core_axis_name="c", subcore_axis_name="s")

_GDN = lax.GatherDimensionNumbers(offset_dims=(), collapsed_slice_dims=(0,),
                                  start_index_map=(0,))


def _perm(x, idx):
    return lax.gather(x, idx.reshape(_L, 1), _GDN, (1,),
                      mode=lax.GatherScatterMode.PROMISE_IN_BOUNDS)


def _allreduce_argmax(val, idx, lane):
    for d in (1, 2, 4, 8):
        p = lane ^ d
        oval = _perm(val, p)
        oidx = _perm(idx, p)
        take = (oval > val) | ((oval == val) & (oidx < idx))
        val = jnp.where(take, oval, val)
        idx = jnp.where(take, oidx, idx)
    return val, idx


@functools.partial(
    pl.kernel, mesh=_sc_mesh,
    out_type=jax.ShapeDtypeStruct((2, NUM_NODES, K), jnp.int32),
    scratch_types=[
        pltpu.VMEM((_ROWS_PER_W, NUM_NODES), jnp.float32),
        pltpu.VMEM((NUM_NODES,), jnp.float32),
        pltpu.VMEM((_L,), jnp.float32),
        pltpu.VMEM((NUM_NODES,), jnp.float32),
        pltpu.VMEM((_ROWS_PER_W, K), jnp.int32),
        pltpu.VMEM((_ROWS_PER_W, K), jnp.int32),
    ],
)
def _topk_sc(num_hbm, xn_hbm, src_hbm, edges_hbm, rows_v, xnall_v, xnrow_v,
             simrow_v, nbr_v, src_v):
    wid = lax.axis_index("s") * 2 + lax.axis_index("c")
    base = wid * _ROWS_PER_W
    pltpu.sync_copy(num_hbm.at[pl.ds(base, _ROWS_PER_W)], rows_v)
    pltpu.sync_copy(xn_hbm, xnall_v)
    pltpu.sync_copy(xn_hbm.at[pl.ds(base, _ROWS_PER_W)], xnrow_v)
    pltpu.sync_copy(src_hbm.at[pl.ds(base, _ROWS_PER_W)], src_v)
    lane = lax.iota(jnp.int32, _L)
    neg = jnp.full((_L,), -jnp.inf, jnp.float32)

    for r in range(_ROWS_PER_W):
        r_idx = jnp.full((_L,), r, jnp.int32)
        xn_i = _perm(xnrow_v[...], r_idx)

        def chunk_div(j, _c, r=r, xn_i=xn_i):
            nc = rows_v[r, pl.ds(j * _L, _L)]
            xj = xnall_v[pl.ds(j * _L, _L)]
            simrow_v[pl.ds(j * _L, _L)] = nc / (xn_i * xj)
            return 0

        lax.fori_loop(0, _NCHUNK, chunk_div, 0)

        def k_body(k, carry, lane=lane, neg=neg):
            nbrvec, kvec, lastv, lasti = carry

            def scan_chunk(j, car, lastv=lastv, lasti=lasti):
                am, ai, jb = car
                cv = simrow_v[pl.ds(j * _L, _L)]
                elig = (cv < lastv) | ((cv == lastv) & (jb > lasti))
                upd = elig & (cv > am)
                am = jnp.where(upd, cv, am)
                ai = jnp.where(upd, jb, ai)
                return (am, ai, jb + _L)

            am, ai, _ = lax.fori_loop(
                0, _NCHUNK, scan_chunk,
                (neg, jnp.full((_L,), 1 << 30, jnp.int32), lane))
            gval, gidx = _allreduce_argmax(am, ai, lane)
            nbrvec = jnp.where(lane == kvec, gidx, nbrvec)
            return (nbrvec, kvec + 1, gval, gidx)

        nbrvec, _, _, _ = lax.fori_loop(
            0, K, k_body,
            (jnp.zeros((_L,), jnp.int32), jnp.zeros((_L,), jnp.int32),
             jnp.full((_L,), jnp.inf, jnp.float32),
             jnp.full((_L,), -1, jnp.int32)))
        nbr_v[r, :] = nbrvec

    pltpu.sync_copy(src_v, edges_hbm.at[0, pl.ds(base, _ROWS_PER_W)])
    pltpu.sync_copy(nbr_v, edges_hbm.at[1, pl.ds(base, _ROWS_PER_W)])


_BB = 16
_NSTEP = BATCH // _BB


def _fused_body(x_ref, wtu_ref, bu_ref, gu_ref, btu_ref, wti_ref, bi_ref,
                gi_ref, bti_ref, hvu_ref, hvi_ref,
                out_ref,
                mu_ref, su_ref, mi_ref, si_ref,
                wsu_ref, biasu_ref, wsi_ref, biasi_ref, xbf_ref):
    t = pl.program_id(0)
    rows = _BB * N_USER

    @pl.when(t < _NSTEP)
    def _stats():
        xu = x_ref[:, 0].reshape(rows, SEQ_LEN)
        xi = x_ref[:, 1].reshape(rows, SEQ_LEN)
        xub = xu.astype(jnp.bfloat16)
        xib = xi.astype(jnp.bfloat16)
        xbf_ref[0, pl.ds(t * rows, rows), :] = xub
        xbf_ref[1, pl.ds(t * rows, rows), :] = xib
        mu = lax.dot_general(xub, xub, (((0,), (0,)), ((), ())),
                             preferred_element_type=jnp.float32)
        mi = lax.dot_general(xib, xib, (((0,), (0,)), ((), ())),
                             preferred_element_type=jnp.float32)
        su = jnp.sum(xu, axis=0, keepdims=True)
        si = jnp.sum(xi, axis=0, keepdims=True)

        @pl.when(t == 0)
        def _():
            mu_ref[...] = mu
            su_ref[...] = su
            mi_ref[...] = mi
            si_ref[...] = si

        @pl.when(t != 0)
        def _():
            mu_ref[...] += mu
            su_ref[...] += su
            mi_ref[...] += mi
            si_ref[...] += si

        @pl.when(t == _NSTEP - 1)
        def _fold():
            def fold(wt_ref, b_ref, g_ref, bt_ref, m_ref, s_ref,
                     ws_ref, bias_ref):
                wt = wt_ref[...]
                bb = b_ref[...]
                g = g_ref[...]
                bt = bt_ref[...]
                ssum = s_ref[...]
                sh = _nn_dot(ssum, wt)
                meanh = sh / N_ROWS + bb
                t2 = _nn_dot(m_ref[...], wt)
                q = jnp.sum(t2 * wt, axis=0, keepdims=True)
                varh = (q + 2.0 * bb * sh) / N_ROWS + bb * bb - meanh * meanh
                sc = g / jnp.sqrt(varh + 1e-5)
                ws_ref[...] = (wt * sc).astype(jnp.bfloat16)
                bias_ref[...] = (bb - meanh) * sc + bt

            fold(wtu_ref, bu_ref, gu_ref, btu_ref, mu_ref, su_ref,
                 wsu_ref, biasu_ref)
            fold(wti_ref, bi_ref, gi_ref, bti_ref, mi_ref, si_ref,
                 wsi_ref, biasi_ref)

    @pl.when(t >= _NSTEP)
    def _main():
        s = (t - _NSTEP) % 4
        b = (t - _NSTEP) // 4

        @pl.when(s == 0)
        def _():
            xm = xbf_ref[0, pl.ds(b * rows, rows), :]
            out_ref[...] = _nn_dot(xm, wsu_ref[...]) + biasu_ref[...]

        @pl.when(s == 1)
        def _():
            hv = hvu_ref[...]
            for i in range(_BB):
                out_ref[pl.ds(i * N_USER, N_USER), :] = hv

        @pl.when(s == 2)
        def _():
            xm = xbf_ref[1, pl.ds(b * rows, rows), :]
            out_ref[...] = _nn_dot(xm, wsi_ref[...]) + biasi_ref[...]

        @pl.when(s == 3)
        def _():
            hv = hvi_ref[...]
            for i in range(_BB):
                out_ref[pl.ds(i * N_USER, N_USER), :] = hv


def kernel(x, emb_table, emb_gamma, emb_beta, xw_u, xb_u, xg_u, xbt_u, vw_u,
           vb_u, vg_u, vbt_u, xw_i, xb_i, xg_i, xbt_i, vw_i, vb_i, vg_i,
           vbt_i):
    f32 = jnp.float32
    row = lambda a: a.reshape(1, D_HIDDEN)

    num, xn, hv_u, hv_i = pl.pallas_call(
        _prep_body,
        out_shape=[
            jax.ShapeDtypeStruct((NUM_NODES, NUM_NODES), f32),
            jax.ShapeDtypeStruct((NUM_NODES, 1), f32),
            jax.ShapeDtypeStruct((N_USER, D_HIDDEN), f32),
            jax.ShapeDtypeStruct((N_USER, D_HIDDEN), f32),
        ],
    )(emb_table, row(emb_gamma), row(emb_beta),
      vw_u, row(vb_u), row(vg_u), row(vbt_u),
      vw_i, row(vb_i), row(vg_i), row(vbt_i))

    src_const = jnp.broadcast_to(
        jnp.arange(NUM_NODES, dtype=jnp.int32)[:, None], (NUM_NODES, K))
    edges3 = _topk_sc(num, xn.reshape(NUM_NODES), src_const)
    edges = edges3.reshape(2, NUM_NODES * K)

    x4 = x.reshape(BATCH, 2, N_USER, SEQ_LEN)
    n4 = 4 * _NSTEP
    cmap = lambda t: (0, 0)

    def x_map(t):
        return (jnp.minimum(t, _NSTEP - 1), 0, 0, 0)

    def out_map(t):
        tm = jnp.maximum(t - _NSTEP, 0)
        return (jnp.where(t < _NSTEP, 0, (tm % 4) * _NSTEP + tm // 4), 0)

    out = pl.pallas_call(
        _fused_body,
        grid=(_NSTEP + n4,),
        in_specs=[
            pl.BlockSpec((_BB, 2, N_USER, SEQ_LEN), x_map),
            pl.BlockSpec((SEQ_LEN, D_HIDDEN), cmap),
            pl.BlockSpec((1, D_HIDDEN), cmap),
            pl.BlockSpec((1, D_HIDDEN), cmap),
            pl.BlockSpec((1, D_HIDDEN), cmap),
            pl.BlockSpec((SEQ_LEN, D_HIDDEN), cmap),
            pl.BlockSpec((1, D_HIDDEN), cmap),
            pl.BlockSpec((1, D_HIDDEN), cmap),
            pl.BlockSpec((1, D_HIDDEN), cmap),
            pl.BlockSpec((N_USER, D_HIDDEN), cmap),
            pl.BlockSpec((N_USER, D_HIDDEN), cmap),
        ],
        out_specs=pl.BlockSpec((_BB * N_USER, D_HIDDEN), out_map),
        out_shape=jax.ShapeDtypeStruct((4 * N_ROWS, D_HIDDEN), f32),
        scratch_shapes=[
            pltpu.VMEM((SEQ_LEN, SEQ_LEN), f32),
            pltpu.VMEM((1, SEQ_LEN), f32),
            pltpu.VMEM((SEQ_LEN, SEQ_LEN), f32),
            pltpu.VMEM((1, SEQ_LEN), f32),
            pltpu.VMEM((SEQ_LEN, D_HIDDEN), jnp.bfloat16),
            pltpu.VMEM((1, D_HIDDEN), f32),
            pltpu.VMEM((SEQ_LEN, D_HIDDEN), jnp.bfloat16),
            pltpu.VMEM((1, D_HIDDEN), f32),
            pltpu.VMEM((2, N_ROWS, SEQ_LEN), jnp.bfloat16),
        ],
    )(x4, jnp.swapaxes(xw_u, 0, 1), row(xb_u), row(xg_u), row(xbt_u),
      jnp.swapaxes(xw_i, 0, 1), row(xb_i), row(xg_i), row(xbt_i),
      hv_u, hv_i)

    return (out, edges)
